# baseline (speedup 1.0000x reference)
import jax
import jax.numpy as jnp
from jax import lax
from jax.experimental import pallas as pl
from jax.experimental.pallas import tpu as pltpu
from jax.experimental.pallas import tpu_sc as plsc

VOCAB = 1000000
EMBED = 64
BATCH = 16384
HIST = 200
PAD_VALUE = 0

NC = 2
NS = 16
NW = NC * NS
BPW = BATCH // NW
RG = 64
G = BPW // RG
CV = EMBED // 16
CR = RG // 16
PB = 5
NB = 2 * PB
CK = 4
NCK = RG // CK


def _sc_body(idx_hbm, table_hbm, out_hbm, idx_v, buf_v, acc_v, e0_v, a_v, b_v,
             *sems):
    c = lax.axis_index("c")
    s = lax.axis_index("s")
    wid = s * NC + c

    pltpu.sync_copy(table_hbm.at[pl.ds(0, 1)], e0_v)
    zero = jnp.zeros((16,), jnp.float32)

    def gather(p, sl):
        return pltpu.make_async_copy(table_hbm.at[idx_v.at[p]], buf_v.at[sl],
                                     sems[sl])

    def group(g, _):
        row0 = wid * BPW + g * RG
        pltpu.sync_copy(idx_hbm.at[wid, g], idx_v)

        @plsc.parallel_loop(0, RG, unroll=8)
        def _zrow(j):
            for cc in range(CV):
                acc_v[j, pl.ds(cc * 16, 16)] = zero

        for sl in range(NB):
            gather(sl, sl).start()

        def consume_block(p0, slots):
            for i, sl in enumerate(slots):
                gather(p0 + i, sl).wait()

            def chunk(ck, _):
                r0 = ck * CK
                acc = [[acc_v[r0 + r, pl.ds(cc * 16, 16)]
                        for cc in range(CV)] for r in range(CK)]
                for sl in slots:
                    for r in range(CK):
                        for cc in range(CV):
                            acc[r][cc] = (acc[r][cc] +
                                          buf_v[sl, r0 + r, pl.ds(cc * 16, 16)])
                for r in range(CK):
                    for cc in range(CV):
                        acc_v[r0 + r, pl.ds(cc * 16, 16)] = acc[r][cc]
                return 0
            lax.fori_loop(0, NCK, chunk, 0)

            for i, sl in enumerate(slots):
                p = p0 + i + NB

                @pl.when(p < HIST)
                def _fire():
                    gather(p, sl).start()

        def blockpair(q, _):
            consume_block(q * NB, tuple(range(PB)))
            consume_block(q * NB + PB, tuple(range(PB, NB)))
            return 0
        lax.fori_loop(0, HIST // NB, blockpair, 0)

        def count(p, cnt):
            out = []
            for c8 in range(CR):
                v = idx_v[p, pl.ds(c8 * 16, 16)]
                out.append(cnt[c8] + jnp.where(v != PAD_VALUE, 1.0, 0.0))
            return tuple(out)
        cnt = lax.fori_loop(0, HIST, count,
                            tuple(zero for _ in range(CR)), unroll=4)

        for c8 in range(CR):
            a = 1.0 / (cnt[c8] + 1e-8)
            b = (cnt[c8] - float(HIST)) * a
            a_v[pl.ds(c8 * 16, 16)] = a
            b_v[pl.ds(c8 * 16, 16)] = b

        e0 = [e0_v[0, pl.ds(cc * 16, 16)] for cc in range(CV)]

        def frow(j, _):
            ji = jnp.full((16,), 0, jnp.int32) + j
            asp = plsc.load_gather(a_v, [ji])
            bsp = plsc.load_gather(b_v, [ji])
            for cc in range(CV):
                x = acc_v[j, pl.ds(cc * 16, 16)]
                acc_v[j, pl.ds(cc * 16, 16)] = x * asp + e0[cc] * bsp
            return 0
        lax.fori_loop(0, RG, frow, 0, unroll=2)

        pltpu.sync_copy(acc_v, out_hbm.at[pl.ds(row0, RG)])
        return 0

    lax.fori_loop(0, G, group, 0)


@jax.jit
def _run(idx, embeddings):
    mesh = plsc.VectorSubcoreMesh(core_axis_name="c", subcore_axis_name="s")
    fn = pl.kernel(
        _sc_body,
        out_type=jax.ShapeDtypeStruct((BATCH, EMBED), jnp.float32),
        mesh=mesh,
        scratch_types=[
            pltpu.VMEM((HIST, RG), jnp.int32),
            pltpu.VMEM((NB, RG, 2 * EMBED), jnp.float32),
            pltpu.VMEM((RG, EMBED), jnp.float32),
            pltpu.VMEM((1, 2 * EMBED), jnp.float32),
            pltpu.VMEM((RG,), jnp.float32),
            pltpu.VMEM((RG,), jnp.float32),
        ] + [pltpu.SemaphoreType.DMA] * NB,
        compiler_params=pltpu.CompilerParams(use_tc_tiling_on_sc=True,
                                             needs_layout_passes=False),
    )
    return fn(idx, embeddings)


def kernel(inputs, embeddings):
    idx = inputs.astype(jnp.int32).reshape(NW, G, RG, HIST)
    idx = idx.transpose(0, 1, 3, 2)
    emb2 = jnp.pad(embeddings, ((0, 0), (0, EMBED)))
    return _run(idx, emb2)


# speedup vs baseline: 1.2628x; 1.2628x over previous
import jax
import jax.numpy as jnp
from jax import lax
from jax.experimental import pallas as pl
from jax.experimental.pallas import tpu as pltpu
from jax.experimental.pallas import tpu_sc as plsc

VOCAB = 1000000
EMBED = 64
BATCH = 16384
HIST = 200
PAD_VALUE = 0

NC = 2
NS = 16
NW = NC * NS
BPW = BATCH // NW
RG = 128
G = BPW // RG
CV = EMBED // 16
CR = RG // 16
PB = 5
NB = 2 * PB
CK = 4
NCK = RG // CK


def _sc_body(idx_hbm, table_hbm, out_hbm, idx_v, buf_v, acc_v, e0_v, a_v, b_v,
             *sems):
    c = lax.axis_index("c")
    s = lax.axis_index("s")
    wid = s * NC + c

    pltpu.sync_copy(table_hbm.at[pl.ds(0, 1)], e0_v)
    zero = jnp.zeros((16,), jnp.float32)

    def gather(p, sl):
        return pltpu.make_async_copy(table_hbm.at[idx_v.at[p]], buf_v.at[sl],
                                     sems[sl])

    def group(g, _):
        row0 = wid * BPW + g * RG
        pltpu.sync_copy(idx_hbm.at[wid, g], idx_v)

        @plsc.parallel_loop(0, RG, unroll=8)
        def _zrow(j):
            for cc in range(CV):
                acc_v[j, pl.ds(cc * 16, 16)] = zero

        for sl in range(NB):
            gather(sl, sl).start()

        def consume_block(p0, slots):
            for i, sl in enumerate(slots):
                gather(p0 + i, sl).wait()

            def chunk(ck, _):
                r0 = ck * CK
                acc = [[acc_v[r0 + r, pl.ds(cc * 16, 16)]
                        for cc in range(CV)] for r in range(CK)]
                for sl in slots:
                    for r in range(CK):
                        for cc in range(CV):
                            acc[r][cc] = (acc[r][cc] +
                                          buf_v[sl, r0 + r, pl.ds(cc * 16, 16)])
                for r in range(CK):
                    for cc in range(CV):
                        acc_v[r0 + r, pl.ds(cc * 16, 16)] = acc[r][cc]
                return 0
            lax.fori_loop(0, NCK, chunk, 0)

            for i, sl in enumerate(slots):
                p = p0 + i + NB

                @pl.when(p < HIST)
                def _fire():
                    gather(p, sl).start()

        def blockpair(q, _):
            consume_block(q * NB, tuple(range(PB)))
            consume_block(q * NB + PB, tuple(range(PB, NB)))
            return 0
        lax.fori_loop(0, HIST // NB, blockpair, 0)

        def count(p, cnt):
            out = []
            for c8 in range(CR):
                v = idx_v[p, pl.ds(c8 * 16, 16)]
                out.append(cnt[c8] + jnp.where(v != PAD_VALUE, 1.0, 0.0))
            return tuple(out)
        cnt = lax.fori_loop(0, HIST, count,
                            tuple(zero for _ in range(CR)), unroll=4)

        for c8 in range(CR):
            a = jnp.where(cnt[c8] > 0.0, 1.0 / (cnt[c8] + 1e-8), 0.0)
            b = (cnt[c8] - float(HIST)) * a
            a_v[pl.ds(c8 * 16, 16)] = a
            b_v[pl.ds(c8 * 16, 16)] = b

        e0 = [e0_v[0, pl.ds(cc * 16, 16)] for cc in range(CV)]

        def frow(j, _):
            ji = jnp.full((16,), 0, jnp.int32) + j
            asp = plsc.load_gather(a_v, [ji])
            bsp = plsc.load_gather(b_v, [ji])
            for cc in range(CV):
                x = acc_v[j, pl.ds(cc * 16, 16)]
                acc_v[j, pl.ds(cc * 16, 16)] = x * asp + e0[cc] * bsp
            return 0
        lax.fori_loop(0, RG, frow, 0, unroll=2)

        pltpu.sync_copy(acc_v, out_hbm.at[pl.ds(row0, RG)])
        return 0

    lax.fori_loop(0, G, group, 0)


@jax.jit
def _run(idx, embeddings):
    mesh = plsc.VectorSubcoreMesh(core_axis_name="c", subcore_axis_name="s")
    fn = pl.kernel(
        _sc_body,
        out_type=jax.ShapeDtypeStruct((BATCH, EMBED), jnp.float32),
        mesh=mesh,
        scratch_types=[
            pltpu.VMEM((HIST, RG), jnp.int32),
            pltpu.VMEM((NB, RG, EMBED), jnp.float32),
            pltpu.VMEM((RG, EMBED), jnp.float32),
            pltpu.VMEM((1, EMBED), jnp.float32),
            pltpu.VMEM((RG,), jnp.float32),
            pltpu.VMEM((RG,), jnp.float32),
        ] + [pltpu.SemaphoreType.DMA] * NB,
        compiler_params=pltpu.CompilerParams(use_tc_tiling_on_sc=False,
                                             needs_layout_passes=False),
    )
    return fn(idx, embeddings)


def kernel(inputs, embeddings):
    idx = inputs.astype(jnp.int32).reshape(NW, G, RG, HIST)
    idx = idx.transpose(0, 1, 3, 2)
    return _run(idx, embeddings)
